# baseline (speedup 1.0000x reference)
import functools

import jax
import jax.numpy as jnp
from jax import lax
from jax.experimental import pallas as pl
from jax.experimental.pallas import tpu as pltpu
from jax.experimental.pallas import tpu_sc as plsc

F32 = jnp.float32

T = 8
N = 10000
NF = 8
H = 64
C = 13
E = 160000

NC = 2
NS = 16
NW = NC * NS
CHUNK = 128
NCHUNK = 40
EPW = CHUNK * NCHUNK
EPAD = EPW * NW
NPAD = 10016
RPS = NPAD // NS


HA = H + NS


def _enc_body(x_ref, w1t_ref, b1_ref, w2t_ref, b2_ref, out_ref):
    h = jnp.dot(x_ref[...], w1t_ref[...], preferred_element_type=F32)
    h = jnp.maximum(h + b1_ref[...], 0.0)
    out_ref[:, 0:H] = jnp.dot(h, w2t_ref[...], preferred_element_type=F32) + b2_ref[...]
    out_ref[:, H:HA] = jnp.ones((x_ref.shape[0], NS), F32)


def _encoder(x2d, w1t, b1, w2t, b2):
    rows = x2d.shape[0]
    rb = 2000
    grid = rows // rb
    full = lambda a: pl.BlockSpec(a.shape, lambda i: (0,) * a.ndim)
    return pl.pallas_call(
        _enc_body,
        grid=(grid,),
        in_specs=[
            pl.BlockSpec((rb, NF), lambda i: (i, 0)),
            full(w1t), full(b1), full(w2t), full(b2),
        ],
        out_specs=pl.BlockSpec((rb, HA), lambda i: (i, 0)),
        out_shape=jax.ShapeDtypeStruct((rows, HA), F32),
    )(x2d, w1t, b1, w2t, b2)


NB = 2


def _mp_body(h2d, src4, dst4, z80,
             ssum_out,
             idx_s, idx_d, rows_v, z80_v, acc,
             sg0, sg1, ss0, ss1):
    cid = lax.axis_index("c")
    sid = lax.axis_index("s")
    wid = cid * NS + sid
    sg = [sg0, sg1]
    ss = [ss0, ss1]

    pltpu.sync_copy(z80, z80_v)

    def issue_gather(j, b):
        pltpu.async_copy(h2d.at[idx_s.at[j]], rows_v.at[b], sg[b])

    def wait_gather(j, b):
        pltpu.make_async_copy(h2d.at[idx_s.at[j]], rows_v.at[b], sg[b]).wait()

    def issue_scat(j, b):
        pltpu.async_copy(rows_v.at[b], acc.at[idx_d.at[j]], ss[b], add=True)

    def wait_scat(j, b):
        pltpu.make_async_copy(rows_v.at[b], acc.at[idx_d.at[j]], ss[b]).wait()

    def tstep(t, carry):
        pltpu.sync_copy(z80_v, acc.at[pl.ds(sid * RPS, RPS)])
        plsc.subcore_barrier()

        pltpu.sync_copy(src4.at[t, wid], idx_s)
        pltpu.sync_copy(dst4.at[t, wid], idx_d)

        issue_gather(0, 0)
        issue_gather(1, 1)
        wait_gather(0, 0)
        issue_scat(0, 0)

        def mid(i, carry):
            for q in range(2):
                j = 1 + 2 * i + q
                p = (1 + q) % 2
                wait_scat(j - 1, 1 - p)
                issue_gather(j + 1, 1 - p)
                wait_gather(j, p)
                issue_scat(j, p)
            return carry

        lax.fori_loop(0, (NCHUNK - 2) // 2, mid, 0, unroll=False)

        wait_gather(NCHUNK - 1, 1)
        issue_scat(NCHUNK - 1, 1)
        wait_scat(NCHUNK - 2, 0)
        wait_scat(NCHUNK - 1, 1)
        plsc.subcore_barrier()

        pltpu.sync_copy(acc.at[pl.ds(sid * RPS, RPS)],
                        ssum_out.at[t, cid, pl.ds(sid * RPS, RPS)])
        return carry

    lax.fori_loop(0, T, tstep, 0, unroll=False)


def _msgpass(h2d, src4, dst4):
    z80 = jnp.zeros((RPS, HA), F32)
    mesh = plsc.VectorSubcoreMesh(core_axis_name="c", subcore_axis_name="s",
                                  num_cores=NC, num_subcores=NS)
    fn = pl.kernel(
        _mp_body,
        compiler_params=pltpu.CompilerParams(use_tc_tiling_on_sc=False),
        out_type=[
            jax.ShapeDtypeStruct((T, NC, NPAD, HA), F32),
        ],
        mesh=mesh,
        scratch_types=[
            pltpu.VMEM((NCHUNK, CHUNK), jnp.int32),
            pltpu.VMEM((NCHUNK, CHUNK), jnp.int32),
            pltpu.VMEM((NB, CHUNK, HA), F32),
            pltpu.VMEM((RPS, HA), F32),
            pltpu.VMEM_SHARED((NPAD, HA), F32),
            pltpu.SemaphoreType.DMA,
            pltpu.SemaphoreType.DMA,
            pltpu.SemaphoreType.DMA,
            pltpu.SemaphoreType.DMA,
        ],
    )
    return fn(h2d, src4, dst4, z80)[0]


def _sigmoid(x):
    return 1.0 / (1.0 + jnp.exp(-x))


def _temporal_body(hseq, ssum,
                   wih0, whh0, bg0, wih1, whh1, bg1,
                   w_in, b_in, w_out, b_out,
                   p1, pb1, p2, pb2, p3, pb3, out_ref):
    rb = hseq.shape[1]

    h0 = jnp.zeros((rb, H), F32)
    c0 = jnp.zeros((rb, H), F32)
    h1 = jnp.zeros((rb, H), F32)
    c1 = jnp.zeros((rb, H), F32)
    ys = []
    for t in range(T):
        ht = hseq[t][:, 0:H]
        s = ssum[t, 0, :, 0:H] + ssum[t, 1, :, 0:H]
        cn = ssum[t, 0, :, H:H + 1] + ssum[t, 1, :, H:H + 1]
        mean = s / jnp.maximum(cn, 1.0)
        xt = jnp.where(cn > 0.0, (ht + mean) * 0.5, ht)

        g = (jnp.dot(xt, wih0[...], preferred_element_type=F32)
             + jnp.dot(h0, whh0[...], preferred_element_type=F32) + bg0[...])
        ig = _sigmoid(g[:, 0:H])
        fg = _sigmoid(g[:, H:2 * H])
        gg = jnp.tanh(g[:, 2 * H:3 * H])
        og = _sigmoid(g[:, 3 * H:4 * H])
        c0 = fg * c0 + ig * gg
        h0 = og * jnp.tanh(c0)

        g = (jnp.dot(h0, wih1[...], preferred_element_type=F32)
             + jnp.dot(h1, whh1[...], preferred_element_type=F32) + bg1[...])
        ig = _sigmoid(g[:, 0:H])
        fg = _sigmoid(g[:, H:2 * H])
        gg = jnp.tanh(g[:, 2 * H:3 * H])
        og = _sigmoid(g[:, 3 * H:4 * H])
        c1 = fg * c1 + ig * gg
        h1 = og * jnp.tanh(c1)
        ys.append(h1)

    win = w_in[...]
    bin_ = b_in[...]
    q = jnp.dot(ys[T - 1], win[:, 0:H], preferred_element_type=F32) + bin_[:, 0:H]
    ks = [jnp.dot(y, win[:, H:2 * H], preferred_element_type=F32) + bin_[:, H:2 * H]
          for y in ys]
    vs = [jnp.dot(y, win[:, 2 * H:3 * H], preferred_element_type=F32) + bin_[:, 2 * H:3 * H]
          for y in ys]

    dh = H // 4
    o_parts = []
    for hd in range(4):
        lo, hi = hd * dh, (hd + 1) * dh
        qs = q[:, lo:hi]
        sc = jnp.concatenate(
            [jnp.sum(qs * k[:, lo:hi], axis=1, keepdims=True) for k in ks],
            axis=1) * 0.25
        m = jnp.max(sc, axis=1, keepdims=True)
        e = jnp.exp(sc - m)
        att = e / jnp.sum(e, axis=1, keepdims=True)
        o_h = jnp.zeros((rb, dh), F32)
        for t in range(T):
            o_h = o_h + att[:, t:t + 1] * vs[t][:, lo:hi]
        o_parts.append(o_h)
    o = jnp.concatenate(o_parts, axis=1)
    fh = jnp.dot(o, w_out[...], preferred_element_type=F32) + b_out[...]

    z = jnp.maximum(jnp.dot(fh, p1[...], preferred_element_type=F32) + pb1[...], 0.0)
    z = jnp.maximum(jnp.dot(z, p2[...], preferred_element_type=F32) + pb2[...], 0.0)
    out_ref[...] = jnp.dot(z, p3[...], preferred_element_type=F32) + pb3[...]


def _temporal(hseq, ssum, weights):
    rb = 1000
    grid = N // rb
    full = lambda a: pl.BlockSpec(a.shape, lambda i, _n=a.ndim: (0,) * _n)
    in_specs = [
        pl.BlockSpec((T, rb, HA), lambda i: (0, i, 0)),
        pl.BlockSpec((T, NC, rb, HA), lambda i: (0, 0, i, 0)),
    ] + [full(w) for w in weights]
    return pl.pallas_call(
        _temporal_body,
        grid=(grid,),
        in_specs=in_specs,
        out_specs=pl.BlockSpec((rb, C), lambda i: (i, 0)),
        out_shape=jax.ShapeDtypeStruct((N, C), F32),
    )(hseq, ssum, *weights)


def kernel(node_features, edge_index_seq, W1, b1, W2, b2,
           Wih0, Whh0, bih0, bhh0, Wih1, Whh1, bih1, bhh1,
           W_in, b_in, W_out, b_out, P1, pb1, P2, pb2, P3, pb3):
    B = node_features.shape[0]

    x2d = node_features.reshape(T * N, NF)
    h2d = _encoder(x2d, W1.T, b1.reshape(1, H), W2.T, b2.reshape(1, H))

    src = edge_index_seq[:, 0, :]
    dst = edge_index_seq[:, 1, :]
    tshift = (jnp.arange(T, dtype=jnp.int32) * N)[:, None]
    srcp = jnp.concatenate(
        [src, jnp.zeros((T, EPAD - E), jnp.int32)], axis=1) + tshift
    dstp = jnp.concatenate(
        [dst, jnp.full((T, EPAD - E), N, jnp.int32)], axis=1)
    src4 = srcp.reshape(T, NW, NCHUNK, CHUNK)
    dst4 = dstp.reshape(T, NW, NCHUNK, CHUNK)

    ssum = _msgpass(h2d, src4, dst4)

    hseq = h2d.reshape(T, N, HA)
    weights = [
        Wih0.T, Whh0.T, (bih0 + bhh0).reshape(1, 4 * H),
        Wih1.T, Whh1.T, (bih1 + bhh1).reshape(1, 4 * H),
        W_in.T, b_in.reshape(1, 3 * H), W_out.T, b_out.reshape(1, H),
        P1.T, pb1.reshape(1, 2 * H), P2.T, pb2.reshape(1, H),
        P3.T, pb3.reshape(1, C),
    ]
    logits = _temporal(hseq, ssum, weights)
    return logits.reshape(B, N, C)


# speedup vs baseline: 1.0006x; 1.0006x over previous
import functools

import jax
import jax.numpy as jnp
from jax import lax
from jax.experimental import pallas as pl
from jax.experimental.pallas import tpu as pltpu
from jax.experimental.pallas import tpu_sc as plsc

F32 = jnp.float32

T = 8
N = 10000
NF = 8
H = 64
C = 13
E = 160000

NC = 2
NS = 16
NW = NC * NS
CHUNK = 128
NCHUNK = 40
EPW = CHUNK * NCHUNK
EPAD = EPW * NW
NPAD = 10016
RPS = NPAD // NS


HA = H + NS


def _enc_body(x_ref, w1t_ref, b1_ref, w2t_ref, b2_ref, out_ref):
    h = jnp.dot(x_ref[...], w1t_ref[...], preferred_element_type=F32)
    h = jnp.maximum(h + b1_ref[...], 0.0)
    out_ref[:, 0:H] = jnp.dot(h, w2t_ref[...], preferred_element_type=F32) + b2_ref[...]
    out_ref[:, H:HA] = jnp.ones((x_ref.shape[0], NS), F32)


def _encoder(x2d, w1t, b1, w2t, b2):
    rows = x2d.shape[0]
    rb = 2000
    grid = rows // rb
    full = lambda a: pl.BlockSpec(a.shape, lambda i: (0,) * a.ndim)
    return pl.pallas_call(
        _enc_body,
        grid=(grid,),
        in_specs=[
            pl.BlockSpec((rb, NF), lambda i: (i, 0)),
            full(w1t), full(b1), full(w2t), full(b2),
        ],
        out_specs=pl.BlockSpec((rb, HA), lambda i: (i, 0)),
        out_shape=jax.ShapeDtypeStruct((rows, HA), F32),
    )(x2d, w1t, b1, w2t, b2)


NB = 2


def _mp_body(h2d, src4, dst4, z80,
             ssum_out,
             idx_s, idx_d, rows_v, z80_v, acc,
             sg0, sg1):
    cid = lax.axis_index("c")
    sid = lax.axis_index("s")
    wid = cid * NS + sid
    sg = [sg0, sg1]

    pltpu.sync_copy(z80, z80_v)

    def issue_gather(j, b):
        pltpu.async_copy(h2d.at[idx_s.at[j]], rows_v.at[b], sg[b])

    def wait_gather(j, b):
        pltpu.make_async_copy(h2d.at[idx_s.at[j]], rows_v.at[b], sg[b]).wait()

    def issue_scat(j, b):
        pltpu.sync_copy(rows_v.at[b], acc.at[idx_d.at[j]], add=True)

    def tstep(t, carry):
        pltpu.sync_copy(z80_v, acc.at[pl.ds(sid * RPS, RPS)])
        plsc.subcore_barrier()

        pltpu.sync_copy(src4.at[t, wid], idx_s)
        pltpu.sync_copy(dst4.at[t, wid], idx_d)

        issue_gather(0, 0)

        def mid(i, carry):
            for p in range(2):
                j = 2 * i + p
                issue_gather(j + 1, 1 - p)
                wait_gather(j, p)
                issue_scat(j, p)
            return carry

        lax.fori_loop(0, (NCHUNK - 2) // 2, mid, 0, unroll=False)

        issue_gather(NCHUNK - 1, 1)
        wait_gather(NCHUNK - 2, 0)
        issue_scat(NCHUNK - 2, 0)
        wait_gather(NCHUNK - 1, 1)
        issue_scat(NCHUNK - 1, 1)
        plsc.subcore_barrier()

        base = (t * NC + cid) * NPAD + sid * RPS
        pltpu.sync_copy(acc.at[pl.ds(sid * RPS, RPS)],
                        ssum_out.at[pl.ds(base, RPS)])
        return carry

    lax.fori_loop(0, T, tstep, 0, unroll=False)


def _msgpass(h2d, src4, dst4):
    z80 = jnp.zeros((RPS, HA), F32)
    mesh = plsc.VectorSubcoreMesh(core_axis_name="c", subcore_axis_name="s",
                                  num_cores=NC, num_subcores=NS)
    fn = pl.kernel(
        _mp_body,
        compiler_params=pltpu.CompilerParams(use_tc_tiling_on_sc=False),
        out_type=[
            jax.ShapeDtypeStruct((T * NC * NPAD, HA), F32),
        ],
        mesh=mesh,
        scratch_types=[
            pltpu.VMEM((NCHUNK, CHUNK), jnp.int32),
            pltpu.VMEM((NCHUNK, CHUNK), jnp.int32),
            pltpu.VMEM((NB, CHUNK, HA), F32),
            pltpu.VMEM((RPS, HA), F32),
            pltpu.VMEM_SHARED((NPAD, HA), F32),
            pltpu.SemaphoreType.DMA,
            pltpu.SemaphoreType.DMA,
        ],
    )
    return fn(h2d, src4, dst4, z80)[0].reshape(T, NC, NPAD, HA)


def _sigmoid(x):
    return 1.0 / (1.0 + jnp.exp(-x))


def _temporal_body(hseq, ssum,
                   wih0, whh0, bg0, wih1, whh1, bg1,
                   w_in, b_in, w_out, b_out,
                   p1, pb1, p2, pb2, p3, pb3, out_ref):
    rb = hseq.shape[1]

    h0 = jnp.zeros((rb, H), F32)
    c0 = jnp.zeros((rb, H), F32)
    h1 = jnp.zeros((rb, H), F32)
    c1 = jnp.zeros((rb, H), F32)
    ys = []
    for t in range(T):
        ht = hseq[t][:, 0:H]
        s = ssum[t, 0, :, 0:H] + ssum[t, 1, :, 0:H]
        cn = ssum[t, 0, :, H:H + 1] + ssum[t, 1, :, H:H + 1]
        mean = s / jnp.maximum(cn, 1.0)
        xt = jnp.where(cn > 0.0, (ht + mean) * 0.5, ht)

        g = (jnp.dot(xt, wih0[...], preferred_element_type=F32)
             + jnp.dot(h0, whh0[...], preferred_element_type=F32) + bg0[...])
        ig = _sigmoid(g[:, 0:H])
        fg = _sigmoid(g[:, H:2 * H])
        gg = jnp.tanh(g[:, 2 * H:3 * H])
        og = _sigmoid(g[:, 3 * H:4 * H])
        c0 = fg * c0 + ig * gg
        h0 = og * jnp.tanh(c0)

        g = (jnp.dot(h0, wih1[...], preferred_element_type=F32)
             + jnp.dot(h1, whh1[...], preferred_element_type=F32) + bg1[...])
        ig = _sigmoid(g[:, 0:H])
        fg = _sigmoid(g[:, H:2 * H])
        gg = jnp.tanh(g[:, 2 * H:3 * H])
        og = _sigmoid(g[:, 3 * H:4 * H])
        c1 = fg * c1 + ig * gg
        h1 = og * jnp.tanh(c1)
        ys.append(h1)

    win = w_in[...]
    bin_ = b_in[...]
    q = jnp.dot(ys[T - 1], win[:, 0:H], preferred_element_type=F32) + bin_[:, 0:H]
    ks = [jnp.dot(y, win[:, H:2 * H], preferred_element_type=F32) + bin_[:, H:2 * H]
          for y in ys]
    vs = [jnp.dot(y, win[:, 2 * H:3 * H], preferred_element_type=F32) + bin_[:, 2 * H:3 * H]
          for y in ys]

    dh = H // 4
    o_parts = []
    for hd in range(4):
        lo, hi = hd * dh, (hd + 1) * dh
        qs = q[:, lo:hi]
        sc = jnp.concatenate(
            [jnp.sum(qs * k[:, lo:hi], axis=1, keepdims=True) for k in ks],
            axis=1) * 0.25
        m = jnp.max(sc, axis=1, keepdims=True)
        e = jnp.exp(sc - m)
        att = e / jnp.sum(e, axis=1, keepdims=True)
        o_h = jnp.zeros((rb, dh), F32)
        for t in range(T):
            o_h = o_h + att[:, t:t + 1] * vs[t][:, lo:hi]
        o_parts.append(o_h)
    o = jnp.concatenate(o_parts, axis=1)
    fh = jnp.dot(o, w_out[...], preferred_element_type=F32) + b_out[...]

    z = jnp.maximum(jnp.dot(fh, p1[...], preferred_element_type=F32) + pb1[...], 0.0)
    z = jnp.maximum(jnp.dot(z, p2[...], preferred_element_type=F32) + pb2[...], 0.0)
    out_ref[...] = jnp.dot(z, p3[...], preferred_element_type=F32) + pb3[...]


def _temporal(hseq, ssum, weights):
    rb = 1000
    grid = N // rb
    full = lambda a: pl.BlockSpec(a.shape, lambda i, _n=a.ndim: (0,) * _n)
    in_specs = [
        pl.BlockSpec((T, rb, HA), lambda i: (0, i, 0)),
        pl.BlockSpec((T, NC, rb, HA), lambda i: (0, 0, i, 0)),
    ] + [full(w) for w in weights]
    return pl.pallas_call(
        _temporal_body,
        grid=(grid,),
        in_specs=in_specs,
        out_specs=pl.BlockSpec((rb, C), lambda i: (i, 0)),
        out_shape=jax.ShapeDtypeStruct((N, C), F32),
    )(hseq, ssum, *weights)


def kernel(node_features, edge_index_seq, W1, b1, W2, b2,
           Wih0, Whh0, bih0, bhh0, Wih1, Whh1, bih1, bhh1,
           W_in, b_in, W_out, b_out, P1, pb1, P2, pb2, P3, pb3):
    B = node_features.shape[0]

    x2d = node_features.reshape(T * N, NF)
    h2d = _encoder(x2d, W1.T, b1.reshape(1, H), W2.T, b2.reshape(1, H))

    src = edge_index_seq[:, 0, :]
    dst = edge_index_seq[:, 1, :]
    tshift = (jnp.arange(T, dtype=jnp.int32) * N)[:, None]
    srcp = jnp.concatenate(
        [src, jnp.zeros((T, EPAD - E), jnp.int32)], axis=1) + tshift
    dstp = jnp.concatenate(
        [dst, jnp.full((T, EPAD - E), N, jnp.int32)], axis=1)
    src4 = srcp.reshape(T, NW, NCHUNK, CHUNK)
    dst4 = dstp.reshape(T, NW, NCHUNK, CHUNK)

    ssum = _msgpass(h2d, src4, dst4)

    hseq = h2d.reshape(T, N, HA)
    weights = [
        Wih0.T, Whh0.T, (bih0 + bhh0).reshape(1, 4 * H),
        Wih1.T, Whh1.T, (bih1 + bhh1).reshape(1, 4 * H),
        W_in.T, b_in.reshape(1, 3 * H), W_out.T, b_out.reshape(1, H),
        P1.T, pb1.reshape(1, 2 * H), P2.T, pb2.reshape(1, H),
        P3.T, pb3.reshape(1, C),
    ]
    logits = _temporal(hseq, ssum, weights)
    return logits.reshape(B, N, C)


# speedup vs baseline: 1.2330x; 1.2323x over previous
import functools

import jax
import jax.numpy as jnp
from jax import lax
from jax.experimental import pallas as pl
from jax.experimental.pallas import tpu as pltpu
from jax.experimental.pallas import tpu_sc as plsc

F32 = jnp.float32

T = 8
N = 10000
NF = 8
H = 64
C = 13
E = 160000

NC = 2
NS = 16
NW = NC * NS
CHUNK = 128
NCHUNK = 40
EPW = CHUNK * NCHUNK
EPAD = EPW * NW
NPAD = 10016
RPS = NPAD // NS


def _enc_body(x_ref, w1t_ref, b1_ref, w2t_ref, b2_ref, out_ref):
    h = jnp.dot(x_ref[...], w1t_ref[...], preferred_element_type=F32)
    h = jnp.maximum(h + b1_ref[...], 0.0)
    out_ref[...] = jnp.dot(h, w2t_ref[...], preferred_element_type=F32) + b2_ref[...]


def _encoder(x2d, w1t, b1, w2t, b2):
    rows = x2d.shape[0]
    rb = 2000
    grid = rows // rb
    full = lambda a: pl.BlockSpec(a.shape, lambda i: (0,) * a.ndim)
    return pl.pallas_call(
        _enc_body,
        grid=(grid,),
        in_specs=[
            pl.BlockSpec((rb, NF), lambda i: (i, 0)),
            full(w1t), full(b1), full(w2t), full(b2),
        ],
        out_specs=pl.BlockSpec((rb, H), lambda i: (i, 0)),
        out_shape=jax.ShapeDtypeStruct((rows, H), F32),
    )(x2d, w1t, b1, w2t, b2)


NB = 2


RSTG = N // NS


def _mp_body(h2d, src4, dst4, z64, z16, ones16,
             ssum_out, cnt_out,
             idx_s, idx_d, rows_v, z64_v, z16_v, ones_v, acc, cacc,
             sg0, sg1):
    cid = lax.axis_index("c")
    sid = lax.axis_index("s")
    wid = cid * NS + sid
    sg = [sg0, sg1]

    pltpu.sync_copy(z64, z64_v)
    pltpu.sync_copy(z16, z16_v)
    pltpu.sync_copy(ones16, ones_v)

    def issue_gather(j, b):
        pltpu.async_copy(h2d.at[idx_s.at[j]], rows_v.at[b], sg[b])

    def wait_gather(j, b):
        pltpu.make_async_copy(h2d.at[idx_s.at[j]], rows_v.at[b], sg[b]).wait()

    def issue_scat(j, b):
        pltpu.sync_copy(rows_v.at[b], acc.at[idx_d.at[j]], add=True)
        pltpu.sync_copy(ones_v, cacc.at[idx_d.at[j]], add=True)

    def tstep(t, carry):
        pltpu.sync_copy(z64_v, acc.at[pl.ds(sid * RPS, RPS)])
        pltpu.sync_copy(z16_v, cacc.at[pl.ds(sid * RPS, RPS)])
        plsc.subcore_barrier()

        pltpu.sync_copy(src4.at[t, wid], idx_s)
        pltpu.sync_copy(dst4.at[t, wid], idx_d)

        issue_gather(0, 0)

        def mid(i, carry):
            for p in range(2):
                j = 2 * i + p
                issue_gather(j + 1, 1 - p)
                wait_gather(j, p)
                issue_scat(j, p)
            return carry

        lax.fori_loop(0, (NCHUNK - 2) // 2, mid, 0, unroll=False)

        issue_gather(NCHUNK - 1, 1)
        wait_gather(NCHUNK - 2, 0)
        issue_scat(NCHUNK - 2, 0)
        wait_gather(NCHUNK - 1, 1)
        issue_scat(NCHUNK - 1, 1)
        plsc.subcore_barrier()

        pltpu.sync_copy(acc.at[pl.ds(sid * RPS, RPS)],
                        ssum_out.at[t, cid, pl.ds(sid * RPS, RPS)])
        pltpu.sync_copy(cacc.at[pl.ds(sid * RPS, RPS)],
                        cnt_out.at[t, cid, pl.ds(sid * RPS, RPS)])
        return carry

    lax.fori_loop(0, T, tstep, 0, unroll=False)


def _msgpass(h2d, src4, dst4):
    z64 = jnp.zeros((RPS, H), F32)
    z16 = jnp.zeros((RPS, NS), F32)
    ones16 = jnp.ones((CHUNK, NS), F32)
    mesh = plsc.VectorSubcoreMesh(core_axis_name="c", subcore_axis_name="s",
                                  num_cores=NC, num_subcores=NS)
    fn = pl.kernel(
        _mp_body,
        compiler_params=pltpu.CompilerParams(use_tc_tiling_on_sc=False),
        out_type=[
            jax.ShapeDtypeStruct((T, NC, NPAD, H), F32),
            jax.ShapeDtypeStruct((T, NC, NPAD, NS), F32),
        ],
        mesh=mesh,
        scratch_types=[
            pltpu.VMEM((NCHUNK, CHUNK), jnp.int32),
            pltpu.VMEM((NCHUNK, CHUNK), jnp.int32),
            pltpu.VMEM((NB, CHUNK, H), F32),
            pltpu.VMEM((RPS, H), F32),
            pltpu.VMEM((RPS, NS), F32),
            pltpu.VMEM((CHUNK, NS), F32),
            pltpu.VMEM_SHARED((NPAD, H), F32),
            pltpu.VMEM_SHARED((NPAD, NS), F32),
            pltpu.SemaphoreType.DMA,
            pltpu.SemaphoreType.DMA,
        ],
    )
    return fn(h2d, src4, dst4, z64, z16, ones16)


def _sigmoid(x):
    return 1.0 / (1.0 + jnp.exp(-x))


def _temporal_body(hseq, ssum, cnt,
                   wih0, whh0, bg0, wih1, whh1, bg1,
                   w_in, b_in, w_out, b_out,
                   p1, pb1, p2, pb2, p3, pb3, out_ref):
    rb = hseq.shape[1]

    h0 = jnp.zeros((rb, H), F32)
    c0 = jnp.zeros((rb, H), F32)
    h1 = jnp.zeros((rb, H), F32)
    c1 = jnp.zeros((rb, H), F32)
    ys = []
    for t in range(T):
        ht = hseq[t]
        s = ssum[t, 0] + ssum[t, 1]
        cn = cnt[t, 0, :, 0:1] + cnt[t, 1, :, 0:1]
        mean = s / jnp.maximum(cn, 1.0)
        xt = jnp.where(cn > 0.0, (ht + mean) * 0.5, ht)

        g = (jnp.dot(xt, wih0[...], preferred_element_type=F32)
             + jnp.dot(h0, whh0[...], preferred_element_type=F32) + bg0[...])
        ig = _sigmoid(g[:, 0:H])
        fg = _sigmoid(g[:, H:2 * H])
        gg = jnp.tanh(g[:, 2 * H:3 * H])
        og = _sigmoid(g[:, 3 * H:4 * H])
        c0 = fg * c0 + ig * gg
        h0 = og * jnp.tanh(c0)

        g = (jnp.dot(h0, wih1[...], preferred_element_type=F32)
             + jnp.dot(h1, whh1[...], preferred_element_type=F32) + bg1[...])
        ig = _sigmoid(g[:, 0:H])
        fg = _sigmoid(g[:, H:2 * H])
        gg = jnp.tanh(g[:, 2 * H:3 * H])
        og = _sigmoid(g[:, 3 * H:4 * H])
        c1 = fg * c1 + ig * gg
        h1 = og * jnp.tanh(c1)
        ys.append(h1)

    win = w_in[...]
    bin_ = b_in[...]
    q = jnp.dot(ys[T - 1], win[:, 0:H], preferred_element_type=F32) + bin_[:, 0:H]
    ks = [jnp.dot(y, win[:, H:2 * H], preferred_element_type=F32) + bin_[:, H:2 * H]
          for y in ys]
    vs = [jnp.dot(y, win[:, 2 * H:3 * H], preferred_element_type=F32) + bin_[:, 2 * H:3 * H]
          for y in ys]

    dh = H // 4
    o_parts = []
    for hd in range(4):
        lo, hi = hd * dh, (hd + 1) * dh
        qs = q[:, lo:hi]
        sc = jnp.concatenate(
            [jnp.sum(qs * k[:, lo:hi], axis=1, keepdims=True) for k in ks],
            axis=1) * 0.25
        m = jnp.max(sc, axis=1, keepdims=True)
        e = jnp.exp(sc - m)
        att = e / jnp.sum(e, axis=1, keepdims=True)
        o_h = jnp.zeros((rb, dh), F32)
        for t in range(T):
            o_h = o_h + att[:, t:t + 1] * vs[t][:, lo:hi]
        o_parts.append(o_h)
    o = jnp.concatenate(o_parts, axis=1)
    fh = jnp.dot(o, w_out[...], preferred_element_type=F32) + b_out[...]

    z = jnp.maximum(jnp.dot(fh, p1[...], preferred_element_type=F32) + pb1[...], 0.0)
    z = jnp.maximum(jnp.dot(z, p2[...], preferred_element_type=F32) + pb2[...], 0.0)
    out_ref[...] = jnp.dot(z, p3[...], preferred_element_type=F32) + pb3[...]


def _temporal(hseq, ssum, cnt, weights):
    rb = 1000
    grid = N // rb
    full = lambda a: pl.BlockSpec(a.shape, lambda i, _n=a.ndim: (0,) * _n)
    in_specs = [
        pl.BlockSpec((T, rb, H), lambda i: (0, i, 0)),
        pl.BlockSpec((T, NC, rb, H), lambda i: (0, 0, i, 0)),
        pl.BlockSpec((T, NC, rb, NS), lambda i: (0, 0, i, 0)),
    ] + [full(w) for w in weights]
    return pl.pallas_call(
        _temporal_body,
        grid=(grid,),
        in_specs=in_specs,
        out_specs=pl.BlockSpec((rb, C), lambda i: (i, 0)),
        out_shape=jax.ShapeDtypeStruct((N, C), F32),
    )(hseq, ssum, cnt, *weights)


def kernel(node_features, edge_index_seq, W1, b1, W2, b2,
           Wih0, Whh0, bih0, bhh0, Wih1, Whh1, bih1, bhh1,
           W_in, b_in, W_out, b_out, P1, pb1, P2, pb2, P3, pb3):
    B = node_features.shape[0]

    x2d = node_features.reshape(T * N, NF)
    h2d = _encoder(x2d, W1.T, b1.reshape(1, H), W2.T, b2.reshape(1, H))

    src = edge_index_seq[:, 0, :]
    dst = edge_index_seq[:, 1, :]
    tshift = (jnp.arange(T, dtype=jnp.int32) * N)[:, None]
    srcp = jnp.concatenate(
        [src, jnp.zeros((T, EPAD - E), jnp.int32)], axis=1) + tshift
    dstp = jnp.concatenate(
        [dst, jnp.full((T, EPAD - E), N, jnp.int32)], axis=1)
    src4 = srcp.reshape(T, NW, NCHUNK, CHUNK)
    dst4 = dstp.reshape(T, NW, NCHUNK, CHUNK)

    ssum, cnt = _msgpass(h2d, src4, dst4)

    hseq = h2d.reshape(T, N, H)
    weights = [
        Wih0.T, Whh0.T, (bih0 + bhh0).reshape(1, 4 * H),
        Wih1.T, Whh1.T, (bih1 + bhh1).reshape(1, 4 * H),
        W_in.T, b_in.reshape(1, 3 * H), W_out.T, b_out.reshape(1, H),
        P1.T, pb1.reshape(1, 2 * H), P2.T, pb2.reshape(1, H),
        P3.T, pb3.reshape(1, C),
    ]
    logits = _temporal(hseq, ssum, cnt, weights)
    return logits.reshape(B, N, C)
